# Initial kernel scaffold; baseline (speedup 1.0000x reference)
import jax
import jax.numpy as jnp
from jax.experimental import pallas as pl


def kernel(x0, i0, x1, idx1, coeff1, idx2, coeff2, idx3, coeff3):
    raise NotImplementedError("write your pallas kernel here")


# speedup vs baseline: 12.2248x; 12.2248x over previous
import functools

import jax
import jax.numpy as jnp
from jax.experimental import pallas as pl


N_ROWS_BLOCK = 512
Z, X0, X1, X2 = 64, 128, 128, 128


def _onehot_cols(idx, depth, dtype=jnp.float32):
    return (idx[None, :] == jnp.arange(depth, dtype=idx.dtype)[:, None]).astype(dtype)


def _tc_body(x0_ref, i0_ref, x1_ref,
             gw1_ref, gj1_ref, s1_ref,
             gw2_ref, ga2_ref, gb2_ref, s2_ref,
             gw3_ref, ga3_ref, gb3_ref, gc3_ref, s3_ref,
             out_ref):
    f32 = jnp.float32
    x1 = x1_ref[...]
    i0 = i0_ref[0, 0, :]
    oh = (i0[:, None] == jax.lax.broadcasted_iota(jnp.int32, (1, Z), 1)).astype(f32)
    x0g = jnp.dot(oh, x0_ref[...], preferred_element_type=f32)

    def deg(gw, *gjs_and_s):
        gjs, s = gjs_and_s[:-1], gjs_and_s[-1]
        v = jnp.dot(x0g, gw, preferred_element_type=f32)
        for gj in gjs:
            v = v * jnp.dot(x1, gj, preferred_element_type=f32)
        return jnp.dot(v, s, preferred_element_type=f32)

    acc = deg(gw1_ref[...], gj1_ref[...], s1_ref[...])
    acc += deg(gw2_ref[...], ga2_ref[...], gb2_ref[...], s2_ref[...])
    acc += deg(gw3_ref[...], ga3_ref[...], gb3_ref[...], gc3_ref[...], s3_ref[...])
    out_ref[...] = acc


@jax.jit
def kernel(x0, i0, x1, idx1, coeff1, idx2, coeff2, idx3, coeff3):
    n = x1.shape[0]
    nb = pl.cdiv(n, N_ROWS_BLOCK)

    gw1 = _onehot_cols(idx1[:, 0], X0)
    gj1 = _onehot_cols(idx1[:, 1], X1)
    s1 = _onehot_cols(idx1[:, 2], X2).T * coeff1[:, None]
    gw2 = _onehot_cols(idx2[:, 0], X0)
    ga2 = _onehot_cols(idx2[:, 1], X1)
    gb2 = _onehot_cols(idx2[:, 2], X1)
    s2 = _onehot_cols(idx2[:, 3], X2).T * coeff2[:, None]
    gw3 = _onehot_cols(idx3[:, 0], X0)
    ga3 = _onehot_cols(idx3[:, 1], X1)
    gb3 = _onehot_cols(idx3[:, 2], X1)
    gc3 = _onehot_cols(idx3[:, 3], X1)
    s3 = _onehot_cols(idx3[:, 4], X2).T * coeff3[:, None]

    i0_3d = i0.reshape(nb, 1, N_ROWS_BLOCK) if n % N_ROWS_BLOCK == 0 else (
        jnp.pad(i0, (0, nb * N_ROWS_BLOCK - n)).reshape(nb, 1, N_ROWS_BLOCK))

    full = lambda shape: pl.BlockSpec(shape, lambda b: (0,) * len(shape))
    out = pl.pallas_call(
        _tc_body,
        grid=(nb,),
        in_specs=[
            full((Z, X0)),
            pl.BlockSpec((1, 1, N_ROWS_BLOCK), lambda b: (b, 0, 0)),
            pl.BlockSpec((N_ROWS_BLOCK, X1), lambda b: (b, 0)),
            full((X0, 64)), full((X1, 64)), full((64, X2)),
            full((X0, 128)), full((X1, 128)), full((X1, 128)), full((128, X2)),
            full((X0, 256)), full((X1, 256)), full((X1, 256)), full((X1, 256)),
            full((256, X2)),
        ],
        out_specs=pl.BlockSpec((N_ROWS_BLOCK, X2), lambda b: (b, 0)),
        out_shape=jax.ShapeDtypeStruct((n, X2), x1.dtype),
    )(x0, i0_3d, x1,
      gw1, gj1, s1, gw2, ga2, gb2, s2, gw3, ga3, gb3, gc3, s3)
    return out
